# Initial kernel scaffold; baseline (speedup 1.0000x reference)
import jax
import jax.numpy as jnp
from jax.experimental import pallas as pl


def kernel(tokens, table):
    raise NotImplementedError("write your pallas kernel here")


# speedup vs baseline: 1.1041x; 1.1041x over previous
import functools

import jax
import jax.numpy as jnp
from jax import lax
from jax.experimental import pallas as pl
from jax.experimental.pallas import tpu as pltpu
from jax.experimental.pallas import tpu_sc as plsc

D = 32
B = 16384 * 50
NC = 2
NS = 16
NW = NC * NS
BPW = B // NW
CH = 128
NCH = BPW // CH
G = 10
NG = NCH // G


def _make_gather():
  mesh = plsc.VectorSubcoreMesh(core_axis_name="c", subcore_axis_name="s")

  @functools.partial(
      pl.kernel,
      mesh=mesh,
      compiler_params=pltpu.CompilerParams(use_tc_tiling_on_sc=False),
      out_type=jax.ShapeDtypeStruct((B, D), jnp.float32),
      scratch_types=[
          pltpu.VMEM((NCH, CH), jnp.int32),
          pltpu.VMEM((G * CH, D), jnp.float32),
          pltpu.SemaphoreType.DMA,
      ],
  )
  def gather_kernel(tokens_hbm, table_hbm, out_hbm, idx_v, rows_v, gsem):
    wid = lax.axis_index("s") * NC + lax.axis_index("c")
    pltpu.sync_copy(tokens_hbm.at[pl.ds(wid * NCH, NCH)], idx_v)
    base = wid * BPW

    def group(g):
      copies = []
      for j in range(G):
        c = pltpu.make_async_copy(
            table_hbm.at[idx_v.at[g * G + j]],
            rows_v.at[pl.ds(j * CH, CH)],
            gsem,
        )
        c.start()
        copies.append(c)
      for c in copies:
        c.wait()
      pltpu.sync_copy(rows_v, out_hbm.at[pl.ds(base + g * (G * CH), G * CH)])

    pl.loop(0, NG)(group)

  return gather_kernel


_gather = _make_gather()


def kernel(tokens, table):
  flat = tokens.reshape(B // CH, CH)
  out = _gather(flat, table)
  return out.reshape(tokens.shape[0], tokens.shape[1], D)


# speedup vs baseline: 1.1090x; 1.0045x over previous
import functools

import jax
import jax.numpy as jnp
from jax import lax
from jax.experimental import pallas as pl
from jax.experimental.pallas import tpu as pltpu
from jax.experimental.pallas import tpu_sc as plsc

D = 32
B = 16384 * 50
NC = 2
NS = 16
NW = NC * NS
BPW = B // NW
CH = 128
NCH = BPW // CH
G = 10
NG = NCH // G


def _make_gather():
  mesh = plsc.VectorSubcoreMesh(core_axis_name="c", subcore_axis_name="s")

  @functools.partial(
      pl.kernel,
      mesh=mesh,
      compiler_params=pltpu.CompilerParams(use_tc_tiling_on_sc=False),
      out_type=jax.ShapeDtypeStruct((B, D), jnp.float32),
      scratch_types=[
          pltpu.VMEM((NCH, CH), jnp.int32),
          pltpu.VMEM((G * CH, D), jnp.float32),
          pltpu.VMEM((G * CH, D), jnp.float32),
          pltpu.SemaphoreType.DMA,
          pltpu.SemaphoreType.DMA,
          pltpu.SemaphoreType.DMA,
      ],
  )
  def gather_kernel(tokens_hbm, table_hbm, out_hbm, idx_v, buf0, buf1,
                    gsem, ssem0, ssem1):
    wid = lax.axis_index("s") * NC + lax.axis_index("c")
    pltpu.sync_copy(tokens_hbm.at[pl.ds(wid * NCH, NCH)], idx_v)
    base = wid * BPW
    GC = G * CH
    bufs = (buf0, buf1)
    ssems = (ssem0, ssem1)

    def pair(p):
      for b in range(2):
        g = p + b
        buf, ssem = bufs[b], ssems[b]

        @pl.when(p >= 2)
        def _():
          pltpu.make_async_copy(
              buf, out_hbm.at[pl.ds(base + (g - 2) * GC, GC)], ssem
          ).wait()

        copies = []
        for j in range(G):
          c = pltpu.make_async_copy(
              table_hbm.at[idx_v.at[g * G + j]],
              buf.at[pl.ds(j * CH, CH)],
              gsem,
          )
          c.start()
          copies.append(c)
        for c in copies:
          c.wait()
        pltpu.make_async_copy(
            buf, out_hbm.at[pl.ds(base + g * GC, GC)], ssem
        ).start()

    pl.loop(0, NG, step=2)(pair)
    for b in range(2):
      g = NG - 2 + b
      pltpu.make_async_copy(
          bufs[b], out_hbm.at[pl.ds(base + g * GC, GC)], ssems[b]
      ).wait()

  return gather_kernel


_gather = _make_gather()


def kernel(tokens, table):
  flat = tokens.reshape(B // CH, CH)
  out = _gather(flat, table)
  return out.reshape(tokens.shape[0], tokens.shape[1], D)


# speedup vs baseline: 1.1125x; 1.0031x over previous
import functools

import jax
import jax.numpy as jnp
from jax import lax
from jax.experimental import pallas as pl
from jax.experimental.pallas import tpu as pltpu
from jax.experimental.pallas import tpu_sc as plsc

D = 32
B = 16384 * 50
NC = 2
NS = 16
NW = NC * NS
BPW = B // NW
CH = 128
NCH = BPW // CH
G = 10
NG = NCH // G


def _make_gather():
  mesh = plsc.VectorSubcoreMesh(core_axis_name="c", subcore_axis_name="s")

  @functools.partial(
      pl.kernel,
      mesh=mesh,
      compiler_params=pltpu.CompilerParams(use_tc_tiling_on_sc=False),
      out_type=jax.ShapeDtypeStruct((B, D), jnp.float32),
      scratch_types=[
          pltpu.VMEM((NCH, CH), jnp.int32),
          pltpu.VMEM((G * CH, D), jnp.float32),
          pltpu.VMEM((G * CH, D), jnp.float32),
          pltpu.SemaphoreType.DMA,
          pltpu.SemaphoreType.DMA,
          pltpu.SemaphoreType.DMA,
          pltpu.SemaphoreType.DMA,
      ],
  )
  def gather_kernel(tokens_hbm, table_hbm, out_hbm, idx_v, buf0, buf1,
                    gsem0, gsem1, ssem0, ssem1):
    wid = lax.axis_index("s") * NC + lax.axis_index("c")
    pltpu.sync_copy(tokens_hbm.at[pl.ds(wid * NCH, NCH)], idx_v)
    base = wid * BPW
    GC = G * CH
    bufs = (buf0, buf1)
    gsems = (gsem0, gsem1)
    ssems = (ssem0, ssem1)

    def fire(g, buf, gsem):
      for j in range(G):
        pltpu.make_async_copy(
            table_hbm.at[idx_v.at[g * G + j]],
            buf.at[pl.ds(j * CH, CH)],
            gsem,
        ).start()

    def drain_gathers(b):
      pltpu.make_async_copy(
          out_hbm.at[pl.ds(base, GC)], bufs[b], gsems[b]
      ).wait()

    def pair(p):
      for b in range(2):
        g = p + b
        ob = 1 - b

        @pl.when(g >= 2)
        def _():
          pltpu.make_async_copy(
              bufs[b], out_hbm.at[pl.ds(base + (g - 2) * GC, GC)], ssems[b]
          ).wait()

        fire(g, bufs[b], gsems[b])

        @pl.when(g >= 1)
        def _():
          drain_gathers(ob)
          pltpu.make_async_copy(
              bufs[ob], out_hbm.at[pl.ds(base + (g - 1) * GC, GC)], ssems[ob]
          ).start()

    pl.loop(0, NG, step=2)(pair)

    drain_gathers(1)
    final = pltpu.make_async_copy(
        bufs[1], out_hbm.at[pl.ds(base + (NG - 1) * GC, GC)], ssems[1]
    )
    final.start()
    pltpu.make_async_copy(
        bufs[0], out_hbm.at[pl.ds(base + (NG - 2) * GC, GC)], ssems[0]
    ).wait()
    final.wait()

  return gather_kernel


_gather = _make_gather()


def kernel(tokens, table):
  flat = tokens.reshape(B // CH, CH)
  out = _gather(flat, table)
  return out.reshape(tokens.shape[0], tokens.shape[1], D)


# speedup vs baseline: 1.8039x; 1.6215x over previous
import functools

import jax
import jax.numpy as jnp
from jax import lax
from jax.experimental import pallas as pl
from jax.experimental.pallas import tpu as pltpu
from jax.experimental.pallas import tpu_sc as plsc

D = 32
NT = 16384
H = 50
NC = 2
NS = 16
NW = NC * NS
TPW = NT // NW
G = 8
NG = TPW // G


def _make_gather():
  mesh = plsc.VectorSubcoreMesh(core_axis_name="c", subcore_axis_name="s")

  @functools.partial(
      pl.kernel,
      mesh=mesh,
      compiler_params=pltpu.CompilerParams(use_tc_tiling_on_sc=False),
      out_type=jax.ShapeDtypeStruct((NT, H, D), jnp.float32),
      scratch_types=[
          pltpu.VMEM((TPW, H), jnp.int32),
          pltpu.VMEM((G, H, D), jnp.float32),
          pltpu.VMEM((G, H, D), jnp.float32),
          pltpu.SemaphoreType.DMA,
          pltpu.SemaphoreType.DMA,
          pltpu.SemaphoreType.DMA,
          pltpu.SemaphoreType.DMA,
      ],
  )
  def gather_kernel(tokens_hbm, table_hbm, out_hbm, idx_v, buf0, buf1,
                    gsem0, gsem1, ssem0, ssem1):
    wid = lax.axis_index("s") * NC + lax.axis_index("c")
    base = wid * TPW
    pltpu.sync_copy(tokens_hbm.at[pl.ds(base, TPW)], idx_v)
    bufs = (buf0, buf1)
    gsems = (gsem0, gsem1)
    ssems = (ssem0, ssem1)

    def fire(g, buf, gsem):
      for j in range(G):
        pltpu.make_async_copy(
            table_hbm.at[idx_v.at[g * G + j]],
            buf.at[j],
            gsem,
        ).start()

    def drain_gathers(b):
      pltpu.make_async_copy(
          out_hbm.at[pl.ds(base, G)], bufs[b], gsems[b]
      ).wait()

    def pair(p):
      for b in range(2):
        g = p + b
        ob = 1 - b

        @pl.when(g >= 2)
        def _():
          pltpu.make_async_copy(
              bufs[b], out_hbm.at[pl.ds(base + (g - 2) * G, G)], ssems[b]
          ).wait()

        fire(g, bufs[b], gsems[b])

        @pl.when(g >= 1)
        def _():
          drain_gathers(ob)
          pltpu.make_async_copy(
              bufs[ob], out_hbm.at[pl.ds(base + (g - 1) * G, G)], ssems[ob]
          ).start()

    pl.loop(0, NG, step=2)(pair)

    drain_gathers(1)
    final = pltpu.make_async_copy(
        bufs[1], out_hbm.at[pl.ds(base + (NG - 1) * G, G)], ssems[1]
    )
    final.start()
    pltpu.make_async_copy(
        bufs[0], out_hbm.at[pl.ds(base + (NG - 2) * G, G)], ssems[0]
    ).wait()
    final.wait()

  return gather_kernel


_gather = _make_gather()


def kernel(tokens, table):
  return _gather(tokens, table)
